# baseline (speedup 1.0000x reference)
import functools

import jax
import jax.numpy as jnp
from jax import lax
from jax.experimental import pallas as pl
from jax.experimental.pallas import tpu as pltpu
from jax.experimental.pallas import tpu_sc as plsc

N = 10000
D = 128
K = 32
NW = 32
T = 4
NCHUNK = N // T
CPW = 79
NBUF = 5
EPOCHS = (CPW + NBUF - 1) // NBUF
GROUPS = D // 16
INV = 1.0 / (K + 1)
IDXW = T * K


@functools.partial(
    pl.kernel,
    out_type=jax.ShapeDtypeStruct((N, D), jnp.float32),
    mesh=plsc.VectorSubcoreMesh(core_axis_name="c", subcore_axis_name="s"),
    scratch_types=[
        pltpu.VMEM((CPW * IDXW,), jnp.int32),
        [pltpu.VMEM((IDXW, D), jnp.float32)] * NBUF,
        [pltpu.VMEM((T, D), jnp.float32)] * NBUF,
        [pltpu.VMEM((T, D), jnp.float32)] * NBUF,
        [pltpu.SemaphoreType.DMA] * NBUF,
        [pltpu.SemaphoreType.DMA] * NBUF,
        [pltpu.SemaphoreType.DMA] * NBUF,
    ],
)
def _gcn_agg(features_hbm, rowsp_hbm, out_hbm, idx_all, gath, selfb, outb,
             gsem, ssem, osem):
    wid = lax.axis_index("s") * 2 + lax.axis_index("c")
    c0 = wid * CPW

    pltpu.sync_copy(rowsp_hbm.at[pl.ds(c0 * IDXW, CPW * IDXW)], idx_all)

    def valid(j):
        return (j < CPW) & (c0 + j < NCHUNK)

    def gather_pair(j, b):
        src = features_hbm.at[idx_all.at[pl.ds(j * IDXW, IDXW)]]
        return (src, gath[b], gsem[b])

    def issue(j, b):
        @pl.when(valid(j))
        def _():
            pltpu.async_copy(*gather_pair(j, b))
            nb = (c0 + j) * T
            pltpu.async_copy(features_hbm.at[pl.ds(nb, T)], selfb[b], ssem[b])

    def process(j, b):
        @pl.when(valid(j))
        def _():
            nb = (c0 + j) * T
            pltpu.make_async_copy(*gather_pair(j, b)).wait()
            pltpu.make_async_copy(
                features_hbm.at[pl.ds(nb, T)], selfb[b], ssem[b]).wait()
            for t in range(T):
                def kbody(k, accs):
                    return tuple(
                        accs[g] + gath[b][t * K + k, pl.ds(g * 16, 16)]
                        for g in range(GROUPS))
                accs = tuple(
                    selfb[b][t, pl.ds(g * 16, 16)] for g in range(GROUPS))
                accs = lax.fori_loop(0, K, kbody, accs)
                for g in range(GROUPS):
                    outb[b][t, pl.ds(g * 16, 16)] = accs[g] * INV
            pltpu.async_copy(outb[b], out_hbm.at[pl.ds(nb, T)], osem[b])

    def wait_out(j, b):
        @pl.when((j >= 0) & valid(j))
        def _():
            nb = (c0 + j) * T
            pltpu.make_async_copy(outb[b], out_hbm.at[pl.ds(nb, T)],
                                  osem[b]).wait()

    for pj in range(NBUF - 1):
        issue(jnp.int32(pj), pj)

    def body(e, carry):
        for b in range(NBUF):
            j = e * NBUF + b
            issue(j + (NBUF - 1), (b + NBUF - 1) % NBUF)
            wait_out(j - NBUF, b)
            process(j, b)
        return carry

    lax.fori_loop(0, EPOCHS, body, 0)

    for j in range(NBUF * EPOCHS - NBUF, NBUF * EPOCHS):
        wait_out(jnp.int32(j), j % NBUF)


def kernel(features, nodes, rows, num_neighbors):
    del nodes, num_neighbors
    rows_pad = jnp.concatenate(
        [rows.reshape(-1),
         jnp.zeros((NW * CPW - NCHUNK) * IDXW, jnp.int32)])
    return _gcn_agg(features, rows_pad)


# speedup vs baseline: 1.0139x; 1.0139x over previous
import functools

import jax
import jax.numpy as jnp
from jax import lax
from jax.experimental import pallas as pl
from jax.experimental.pallas import tpu as pltpu
from jax.experimental.pallas import tpu_sc as plsc

N = 10000
D = 128
K = 32
NW = 32
T = 4
NCHUNK = N // T
CPW = 79
NBUF = 4
EPOCHS = (CPW + NBUF - 1) // NBUF
GROUPS = D // 16
INV = 1.0 / (K + 1)
IDXW = T * K


@functools.partial(
    pl.kernel,
    out_type=jax.ShapeDtypeStruct((N, D), jnp.float32),
    mesh=plsc.VectorSubcoreMesh(core_axis_name="c", subcore_axis_name="s"),
    scratch_types=[
        pltpu.VMEM((CPW * IDXW,), jnp.int32),
        [pltpu.VMEM((IDXW, D), jnp.float32)] * NBUF,
        [pltpu.VMEM((T, D), jnp.float32)] * NBUF,
        [pltpu.VMEM((T, D), jnp.float32)] * NBUF,
        [pltpu.SemaphoreType.DMA] * NBUF,
        [pltpu.SemaphoreType.DMA] * NBUF,
        [pltpu.SemaphoreType.DMA] * NBUF,
    ],
)
def _gcn_agg(features_hbm, rowsp_hbm, out_hbm, idx_all, gath, selfb, outb,
             gsem, ssem, osem):
    wid = lax.axis_index("s") * 2 + lax.axis_index("c")
    c0 = wid * CPW

    pltpu.sync_copy(rowsp_hbm.at[pl.ds(c0 * IDXW, CPW * IDXW)], idx_all)

    def valid(j):
        return (j < CPW) & (c0 + j < NCHUNK)

    def gather_pair(j, b):
        src = features_hbm.at[idx_all.at[pl.ds(j * IDXW, IDXW)]]
        return (src, gath[b], gsem[b])

    def issue(j, b):
        @pl.when(valid(j))
        def _():
            pltpu.async_copy(*gather_pair(j, b))
            nb = (c0 + j) * T
            pltpu.async_copy(features_hbm.at[pl.ds(nb, T)], selfb[b], ssem[b])

    def process(j, b):
        @pl.when(valid(j))
        def _():
            nb = (c0 + j) * T
            pltpu.make_async_copy(*gather_pair(j, b)).wait()
            pltpu.make_async_copy(
                features_hbm.at[pl.ds(nb, T)], selfb[b], ssem[b]).wait()
            for t in range(T):
                def kbody(k, accs):
                    return tuple(
                        accs[g] + gath[b][t * K + k, pl.ds(g * 16, 16)]
                        for g in range(GROUPS))
                accs = tuple(
                    selfb[b][t, pl.ds(g * 16, 16)] for g in range(GROUPS))
                accs = lax.fori_loop(0, K, kbody, accs)
                for g in range(GROUPS):
                    outb[b][t, pl.ds(g * 16, 16)] = accs[g] * INV
            pltpu.async_copy(outb[b], out_hbm.at[pl.ds(nb, T)], osem[b])

    def wait_out(j, b):
        @pl.when((j >= 0) & valid(j))
        def _():
            nb = (c0 + j) * T
            pltpu.make_async_copy(outb[b], out_hbm.at[pl.ds(nb, T)],
                                  osem[b]).wait()

    for pj in range(NBUF - 1):
        issue(jnp.int32(pj), pj)

    def body(e, carry):
        for b in range(NBUF):
            j = e * NBUF + b
            issue(j + (NBUF - 1), (b + NBUF - 1) % NBUF)
            wait_out(j - NBUF, b)
            process(j, b)
        return carry

    lax.fori_loop(0, EPOCHS, body, 0)

    for j in range(NBUF * EPOCHS - NBUF, NBUF * EPOCHS):
        wait_out(jnp.int32(j), j % NBUF)


def kernel(features, nodes, rows, num_neighbors):
    del nodes, num_neighbors
    rows_pad = jnp.concatenate(
        [rows.reshape(-1),
         jnp.zeros((NW * CPW - NCHUNK) * IDXW, jnp.int32)])
    return _gcn_agg(features, rows_pad)
